# baseline (speedup 1.0000x reference)
import functools

import jax
import jax.numpy as jnp
from jax import lax
from jax.experimental import pallas as pl
from jax.experimental.pallas import tpu as pltpu
from jax.experimental.pallas import tpu_sc as plsc

EMBED = 128
CHUNK = 128
NBUF = 5
PF = 3


@functools.lru_cache(maxsize=None)
def _make_gather(n_rows):
    info = plsc.get_sparse_core_info()
    nw = info.num_cores * info.num_subcores
    rows_per_w = n_rows // nw
    nchunks = rows_per_w // CHUNK
    assert nchunks * CHUNK * nw == n_rows
    assert nchunks % NBUF == 0
    idx_rows_pad = (nchunks + 7) // 8 * 8

    mesh = plsc.VectorSubcoreMesh(core_axis_name="c", subcore_axis_name="s")

    @functools.partial(
        pl.kernel,
        out_type=jax.ShapeDtypeStruct((n_rows, EMBED), jnp.float32),
        mesh=mesh,
        compiler_params=pltpu.CompilerParams(use_tc_tiling_on_sc=True),
        scratch_types=[
            pltpu.VMEM((idx_rows_pad, CHUNK), jnp.int32),
            [pltpu.VMEM((CHUNK, EMBED), jnp.float32)] * NBUF,
            [pltpu.SemaphoreType.DMA] * NBUF,
            [pltpu.SemaphoreType.DMA] * NBUF,
        ],
    )
    def gather_kernel(idx_hbm, table_hbm, out_hbm, idx_v, rows, gsem, ssem):
        wid = lax.axis_index("s") * info.num_cores + lax.axis_index("c")
        pltpu.sync_copy(idx_hbm.at[pl.ds(wid * idx_rows_pad, idx_rows_pad)], idx_v)
        base = wid * rows_per_w

        def gather(j, b):
            return pltpu.async_copy(table_hbm.at[idx_v.at[j]], rows[b], gsem[b])

        def gather_wait(j, b):
            pltpu.make_async_copy(table_hbm.at[idx_v.at[j]], rows[b], gsem[b]).wait()

        def store(j, b):
            dst = out_hbm.at[pl.ds(base + j * CHUNK, CHUNK)]
            return pltpu.async_copy(rows[b], dst, ssem[b])

        def store_wait(b):
            dst = out_hbm.at[pl.ds(base, CHUNK)]
            pltpu.make_async_copy(rows[b], dst, ssem[b]).wait()

        for b in range(PF):
            gather(b, b)

        @pl.loop(0, nchunks // NBUF)
        def _(g):
            j0 = g * NBUF
            for b in range(NBUF):
                j = j0 + b
                jf = j + PF
                bf = (b + PF) % NBUF

                @pl.when(jf >= NBUF)
                def _():
                    store_wait(bf)

                @pl.when(jf < nchunks)
                def _():
                    gather(jf, bf)

                gather_wait(j, b)
                store(j, b)

        for j in range(nchunks - (NBUF - PF), nchunks):
            store_wait(j % NBUF)

    return gather_kernel


def kernel(input, embedding):
    b, s = input.shape
    n_rows = b * s
    info = plsc.get_sparse_core_info()
    nw = info.num_cores * info.num_subcores
    nchunks = n_rows // nw // CHUNK
    pad = (nchunks + 7) // 8 * 8 - nchunks
    idx = jnp.transpose(input).astype(jnp.int32).reshape(nw, nchunks, CHUNK)
    idx = jnp.pad(idx, ((0, 0), (0, pad), (0, 0))).reshape(-1, CHUNK)
    out = _make_gather(n_rows)(idx, embedding)
    return out.reshape(s, b, EMBED).transpose(1, 0, 2)


# speedup vs baseline: 1.0023x; 1.0023x over previous
import functools

import jax
import jax.numpy as jnp
from jax import lax
from jax.experimental import pallas as pl
from jax.experimental.pallas import tpu as pltpu
from jax.experimental.pallas import tpu_sc as plsc

EMBED = 128
CHUNK = 64
NBUF = 10
PF = 5


@functools.lru_cache(maxsize=None)
def _make_gather(n_rows):
    info = plsc.get_sparse_core_info()
    nw = info.num_cores * info.num_subcores
    rows_per_w = n_rows // nw
    nchunks = rows_per_w // CHUNK
    assert nchunks * CHUNK * nw == n_rows
    assert nchunks % NBUF == 0
    idx_rows_pad = (nchunks + 7) // 8 * 8

    mesh = plsc.VectorSubcoreMesh(core_axis_name="c", subcore_axis_name="s")

    @functools.partial(
        pl.kernel,
        out_type=jax.ShapeDtypeStruct((n_rows, EMBED), jnp.float32),
        mesh=mesh,
        compiler_params=pltpu.CompilerParams(use_tc_tiling_on_sc=True),
        scratch_types=[
            pltpu.VMEM((idx_rows_pad, CHUNK), jnp.int32),
            [pltpu.VMEM((CHUNK, EMBED), jnp.float32)] * NBUF,
            [pltpu.SemaphoreType.DMA] * NBUF,
            [pltpu.SemaphoreType.DMA] * NBUF,
        ],
    )
    def gather_kernel(idx_hbm, table_hbm, out_hbm, idx_v, rows, gsem, ssem):
        wid = lax.axis_index("s") * info.num_cores + lax.axis_index("c")
        pltpu.sync_copy(idx_hbm.at[pl.ds(wid * idx_rows_pad, idx_rows_pad)], idx_v)
        base = wid * rows_per_w

        def gather(j, b):
            return pltpu.async_copy(table_hbm.at[idx_v.at[j]], rows[b], gsem[b])

        def gather_wait(j, b):
            pltpu.make_async_copy(table_hbm.at[idx_v.at[j]], rows[b], gsem[b]).wait()

        def store(j, b):
            dst = out_hbm.at[pl.ds(base + j * CHUNK, CHUNK)]
            return pltpu.async_copy(rows[b], dst, ssem[b])

        def store_wait(b):
            dst = out_hbm.at[pl.ds(base, CHUNK)]
            pltpu.make_async_copy(rows[b], dst, ssem[b]).wait()

        for b in range(PF):
            gather(b, b)

        @pl.loop(0, nchunks // NBUF)
        def _(g):
            j0 = g * NBUF
            for b in range(NBUF):
                j = j0 + b
                jf = j + PF
                bf = (b + PF) % NBUF

                @pl.when(jf >= NBUF)
                def _():
                    store_wait(bf)

                @pl.when(jf < nchunks)
                def _():
                    gather(jf, bf)

                gather_wait(j, b)
                store(j, b)

        for j in range(nchunks - (NBUF - PF), nchunks):
            store_wait(j % NBUF)

    return gather_kernel


def kernel(input, embedding):
    b, s = input.shape
    n_rows = b * s
    info = plsc.get_sparse_core_info()
    nw = info.num_cores * info.num_subcores
    nchunks = n_rows // nw // CHUNK
    pad = (nchunks + 7) // 8 * 8 - nchunks
    idx = jnp.transpose(input).astype(jnp.int32).reshape(nw, nchunks, CHUNK)
    idx = jnp.pad(idx, ((0, 0), (0, pad), (0, 0))).reshape(-1, CHUNK)
    out = _make_gather(n_rows)(idx, embedding)
    return out.reshape(s, b, EMBED).transpose(1, 0, 2)
